# baseline (speedup 1.0000x reference)
import functools

import jax
import jax.numpy as jnp
from jax import lax
from jax.experimental import pallas as pl
from jax.experimental.pallas import tpu as pltpu
from jax.experimental.pallas import tpu_sc as plsc

N_NODES = 10000
N_EDGES = 320000
D_IN = 128
D_HID = 64

NC = 2
NS = 16
NW = NC * NS
EDGES_PER_W = N_EDGES // NW
RANGE = 320
N_PAD = NW * RANGE
CAP = 512
GCHUNK = 128
LANES = 16


def _route(src, dst):
    mesh = plsc.VectorSubcoreMesh(core_axis_name="c", subcore_axis_name="s")

    @functools.partial(
        pl.kernel,
        mesh=mesh,
        compiler_params=pltpu.CompilerParams(use_tc_tiling_on_sc=False, needs_layout_passes=False),
        out_type=(
            jax.ShapeDtypeStruct((NW, NW, CAP // GCHUNK, GCHUNK), jnp.int32),
            jax.ShapeDtypeStruct((NW, NW, CAP // GCHUNK, GCHUNK), jnp.int32),
            jax.ShapeDtypeStruct((NW, NW), jnp.int32),
        ),
        scratch_types=[
            pltpu.VMEM((EDGES_PER_W,), jnp.int32),
            pltpu.VMEM((EDGES_PER_W,), jnp.int32),
            pltpu.VMEM((NW, CAP // GCHUNK, GCHUNK), jnp.int32),
            pltpu.VMEM((NW, CAP // GCHUNK, GCHUNK), jnp.int32),
            pltpu.VMEM((NW,), jnp.int32),
            pltpu.VMEM((2 * LANES,), jnp.int32),
        ],
    )
    def k(src_hbm, dst_hbm, ls_hbm, ld_hbm, cnt_hbm,
          sblk, dblk, lsrc, ldst, ctr, shbuf):
        w = lax.axis_index("c") * NS + lax.axis_index("s")
        base = w * EDGES_PER_W
        pltpu.sync_copy(src_hbm.at[pl.ds(base, EDGES_PER_W)], sblk)
        pltpu.sync_copy(dst_hbm.at[pl.ds(base, EDGES_PER_W)], dblk)

        zero = jnp.zeros((LANES,), jnp.int32)
        dump = jnp.full((LANES,), N_PAD, jnp.int32)

        def zbody(i, carry):
            wz = i >> 5
            off = (i & 31) * LANES
            lsrc[wz, off >> 7, pl.ds(off & 127, LANES)] = zero
            ldst[wz, off >> 7, pl.ds(off & 127, LANES)] = dump
            return carry

        lax.fori_loop(0, NW * CAP // LANES, zbody, 0)
        ctr[pl.ds(0, LANES)] = zero
        ctr[pl.ds(LANES, LANES)] = zero
        shbuf[pl.ds(0, LANES)] = jnp.full((LANES,), -1, jnp.int32)

        lane = lax.broadcasted_iota(jnp.int32, (LANES,), 0)
        ones = jnp.ones((LANES,), jnp.int32)

        def body(g, carry):
            e0 = g * LANES
            d16 = dblk[pl.ds(e0, LANES)]
            s16 = sblk[pl.ds(e0, LANES)]
            b16 = ((d16 >> 6) * 6554) >> 15
            shbuf[pl.ds(LANES, LANES)] = b16
            rank = jnp.zeros((LANES,), jnp.int32)
            for sh in range(1, LANES):
                prev = shbuf[pl.ds(LANES - sh, LANES)]
                rank = rank + jnp.where(
                    (prev == b16) & (lane >= sh), ones, zero)
            cbase = plsc.load_gather(ctr, [b16])
            plsc.addupdate_scatter(ctr, [b16], ones)
            q = cbase + rank
            plsc.store_scatter(lsrc, [b16, q >> 7, q & 127], s16)
            plsc.store_scatter(ldst, [b16, q >> 7, q & 127], d16)
            return carry

        lax.fori_loop(0, EDGES_PER_W // LANES, body, 0)
        pltpu.sync_copy(lsrc, ls_hbm.at[w])
        pltpu.sync_copy(ldst, ld_hbm.at[w])
        pltpu.sync_copy(ctr, cnt_hbm.at[w])

    return k(src, dst)


def _ordered_segsum(table, ls, ld, zacc):
    mesh = plsc.VectorSubcoreMesh(core_axis_name="c", subcore_axis_name="s")

    NCH = NW * (CAP // GCHUNK)
    CPW = CAP // GCHUNK

    @functools.partial(
        pl.kernel,
        mesh=mesh,
        compiler_params=pltpu.CompilerParams(use_tc_tiling_on_sc=False),
        out_type=jax.ShapeDtypeStruct((N_PAD, D_HID), jnp.float32),
        scratch_types=[
            pltpu.VMEM((NW, CAP // GCHUNK, GCHUNK), jnp.int32),
            pltpu.VMEM((NW, CAP // GCHUNK, GCHUNK), jnp.int32),
            pltpu.VMEM((GCHUNK, D_HID), jnp.float32),
            pltpu.VMEM((GCHUNK, D_HID), jnp.float32),
            pltpu.VMEM_SHARED((N_PAD + 128, D_HID), jnp.float32),
            pltpu.SemaphoreType.DMA,
            pltpu.SemaphoreType.DMA,
            pltpu.SemaphoreType.DMA,
        ],
    )
    def k(tab_hbm, ls_hbm, ld_hbm, z_hbm, out_hbm,
          lsv, ldv, stage0, stage1, acc, fsem, gsem0, gsem1):
        sid = lax.axis_index("s")
        t = lax.axis_index("c") * NS + sid
        cps = []
        for w in range(NW):
            cps.append(pltpu.async_copy(ls_hbm.at[w, t], lsv.at[w], fsem))
            cps.append(pltpu.async_copy(ld_hbm.at[w, t], ldv.at[w], fsem))
        zrows = (N_PAD + 128) // NS
        pltpu.sync_copy(z_hbm.at[pl.ds(0, zrows)],
                        acc.at[pl.ds(sid * zrows, zrows)])
        for cp in cps:
            cp.wait()
        plsc.subcore_barrier()

        stages = (stage0, stage1)
        sems = (gsem0, gsem1)

        def gidx(kc):
            return lsv.at[kc // CPW, kc % CPW]

        pltpu.async_copy(tab_hbm.at[gidx(0)], stage0, gsem0)

        def kbody(k2, carry):
            for b in range(2):
                kc = k2 * 2 + b
                stage = stages[b]
                pltpu.make_async_copy(tab_hbm.at[gidx(kc)], stage,
                                      sems[b]).wait()

                @pl.when(kc + 1 < NCH)
                def _(kc=kc, b=b):
                    pltpu.async_copy(tab_hbm.at[gidx(kc + 1)],
                                     stages[1 - b], sems[1 - b])

                pltpu.sync_copy(stage, acc.at[ldv.at[kc // CPW, kc % CPW]],
                                add=True)
            return carry

        lax.fori_loop(0, NCH // 2, kbody, 0)
        plsc.subcore_barrier()
        rbase = t * RANGE
        pltpu.sync_copy(acc.at[pl.ds(rbase, RANGE)],
                        out_hbm.at[pl.ds(rbase, RANGE)])

    return k(table, ls, ld, zacc)


def _mm_relu(x, W, b):
    def body(x_ref, w_ref, b_ref, o_ref):
        acc = jnp.dot(x_ref[...], w_ref[...], preferred_element_type=jnp.float32)
        o_ref[...] = jnp.maximum(acc + b_ref[...], 0.0)

    n = x.shape[0]
    return pl.pallas_call(
        body,
        out_shape=jax.ShapeDtypeStruct((n, W.shape[1]), jnp.float32),
    )(x, W, b.reshape(1, -1))


def _final_linear(h2, Wf, bf):
    def body(h_ref, wf_ref, bf_ref, o_ref):
        o_ref[...] = (jnp.dot(h_ref[...], wf_ref[...],
                              preferred_element_type=jnp.float32)
                      + bf_ref[...])

    return pl.pallas_call(
        body,
        out_shape=jax.ShapeDtypeStruct((N_NODES, 1), jnp.float32),
    )(h2, Wf, bf.reshape(1, 1))


def kernel(x, edge_index, W1, b1, W2, b2, Wf, bf):
    src = edge_index[0].astype(jnp.int32)
    dst = edge_index[1].astype(jnp.int32)

    ls, ld, _ = _route(src, dst)
    zacc = jnp.zeros(((N_PAD + 128) // NS, D_HID), jnp.float32)

    m1 = _mm_relu(x, W1, b1)
    h1 = _ordered_segsum(m1, ls, ld, zacc)
    m2 = _mm_relu(h1[:N_NODES], W2, b2)
    h2 = _ordered_segsum(m2, ls, ld, zacc)
    return _final_linear(h2[:N_NODES], Wf, bf)


# speedup vs baseline: 1.0062x; 1.0062x over previous
import functools

import jax
import jax.numpy as jnp
from jax import lax
from jax.experimental import pallas as pl
from jax.experimental.pallas import tpu as pltpu
from jax.experimental.pallas import tpu_sc as plsc

N_NODES = 10000
N_EDGES = 320000
D_IN = 128
D_HID = 64

NC = 2
NS = 16
NW = NC * NS
EDGES_PER_W = N_EDGES // NW
RANGE = 320
N_PAD = NW * RANGE
CAP = 512
GCHUNK = 128
LANES = 16


def _route(src, dst):
    mesh = plsc.VectorSubcoreMesh(core_axis_name="c", subcore_axis_name="s")

    @functools.partial(
        pl.kernel,
        mesh=mesh,
        compiler_params=pltpu.CompilerParams(use_tc_tiling_on_sc=False, needs_layout_passes=False),
        out_type=(
            jax.ShapeDtypeStruct((NW, NW, CAP // GCHUNK, GCHUNK), jnp.int32),
            jax.ShapeDtypeStruct((NW, NW, CAP // GCHUNK, GCHUNK), jnp.int32),
            jax.ShapeDtypeStruct((NW, NW), jnp.int32),
        ),
        scratch_types=[
            pltpu.VMEM((EDGES_PER_W,), jnp.int32),
            pltpu.VMEM((EDGES_PER_W,), jnp.int32),
            pltpu.VMEM((NW, CAP // GCHUNK, GCHUNK), jnp.int32),
            pltpu.VMEM((NW, CAP // GCHUNK, GCHUNK), jnp.int32),
            pltpu.VMEM((NW,), jnp.int32),
            pltpu.VMEM((2 * LANES,), jnp.int32),
        ],
    )
    def k(src_hbm, dst_hbm, ls_hbm, ld_hbm, cnt_hbm,
          sblk, dblk, lsrc, ldst, ctr, shbuf):
        w = lax.axis_index("c") * NS + lax.axis_index("s")
        base = w * EDGES_PER_W
        pltpu.sync_copy(src_hbm.at[pl.ds(base, EDGES_PER_W)], sblk)
        pltpu.sync_copy(dst_hbm.at[pl.ds(base, EDGES_PER_W)], dblk)

        zero = jnp.zeros((LANES,), jnp.int32)
        dump = jnp.full((LANES,), N_PAD, jnp.int32)

        def zbody(i, carry):
            wz = i >> 5
            off = (i & 31) * LANES
            lsrc[wz, off >> 7, pl.ds(off & 127, LANES)] = zero
            ldst[wz, off >> 7, pl.ds(off & 127, LANES)] = dump
            return carry

        lax.fori_loop(0, NW * CAP // LANES, zbody, 0)
        ctr[pl.ds(0, LANES)] = zero
        ctr[pl.ds(LANES, LANES)] = zero
        shbuf[pl.ds(0, LANES)] = jnp.full((LANES,), -1, jnp.int32)

        lane = lax.broadcasted_iota(jnp.int32, (LANES,), 0)
        ones = jnp.ones((LANES,), jnp.int32)

        def body(g, carry):
            e0 = g * LANES
            d16 = dblk[pl.ds(e0, LANES)]
            s16 = sblk[pl.ds(e0, LANES)]
            b16 = d16 & (NW - 1)
            shbuf[pl.ds(LANES, LANES)] = b16
            rank = jnp.zeros((LANES,), jnp.int32)
            for sh in range(1, LANES):
                prev = shbuf[pl.ds(LANES - sh, LANES)]
                rank = rank + jnp.where(
                    (prev == b16) & (lane >= sh), ones, zero)
            cbase = plsc.load_gather(ctr, [b16])
            plsc.addupdate_scatter(ctr, [b16], ones)
            q = cbase + rank
            plsc.store_scatter(lsrc, [b16, q >> 7, q & 127], s16)
            plsc.store_scatter(ldst, [b16, q >> 7, q & 127], d16)
            return carry

        lax.fori_loop(0, EDGES_PER_W // LANES, body, 0)
        pltpu.sync_copy(lsrc, ls_hbm.at[w])
        pltpu.sync_copy(ldst, ld_hbm.at[w])
        pltpu.sync_copy(ctr, cnt_hbm.at[w])

    return k(src, dst)


def _ordered_segsum(table, ls, ld, zacc):
    mesh = plsc.VectorSubcoreMesh(core_axis_name="c", subcore_axis_name="s")

    NCH = NW * (CAP // GCHUNK)
    CPW = CAP // GCHUNK

    @functools.partial(
        pl.kernel,
        mesh=mesh,
        compiler_params=pltpu.CompilerParams(use_tc_tiling_on_sc=False),
        out_type=jax.ShapeDtypeStruct((NC, N_PAD, D_HID), jnp.float32),
        scratch_types=[
            pltpu.VMEM((NW, CAP // GCHUNK, GCHUNK), jnp.int32),
            pltpu.VMEM((NW, CAP // GCHUNK, GCHUNK), jnp.int32),
            pltpu.VMEM((GCHUNK, D_HID), jnp.float32),
            pltpu.VMEM((GCHUNK, D_HID), jnp.float32),
            pltpu.VMEM_SHARED((N_PAD + 128, D_HID), jnp.float32),
            pltpu.SemaphoreType.DMA,
            pltpu.SemaphoreType.DMA,
            pltpu.SemaphoreType.DMA,
        ],
    )
    def k(tab_hbm, ls_hbm, ld_hbm, z_hbm, out_hbm,
          lsv, ldv, stage0, stage1, acc, fsem, gsem0, gsem1):
        sid = lax.axis_index("s")
        t = lax.axis_index("c") * NS + sid
        cps = []
        for w in range(NW):
            cps.append(pltpu.async_copy(ls_hbm.at[w, t], lsv.at[w], fsem))
            cps.append(pltpu.async_copy(ld_hbm.at[w, t], ldv.at[w], fsem))
        zrows = (N_PAD + 128) // NS
        pltpu.sync_copy(z_hbm.at[pl.ds(0, zrows)],
                        acc.at[pl.ds(sid * zrows, zrows)])
        for cp in cps:
            cp.wait()
        plsc.subcore_barrier()

        stages = (stage0, stage1)
        sems = (gsem0, gsem1)

        def gidx(kc):
            return lsv.at[kc // CPW, kc % CPW]

        pltpu.async_copy(tab_hbm.at[gidx(0)], stage0, gsem0)

        def kbody(k2, carry):
            for b in range(2):
                kc = k2 * 2 + b
                stage = stages[b]
                pltpu.make_async_copy(tab_hbm.at[gidx(kc)], stage,
                                      sems[b]).wait()

                @pl.when(kc + 1 < NCH)
                def _(kc=kc, b=b):
                    pltpu.async_copy(tab_hbm.at[gidx(kc + 1)],
                                     stages[1 - b], sems[1 - b])

                pltpu.sync_copy(stage, acc.at[ldv.at[kc // CPW, kc % CPW]],
                                add=True)
            return carry

        lax.fori_loop(0, NCH // 2, kbody, 0)
        plsc.subcore_barrier()
        obase = sid * (N_PAD // NS)
        pltpu.sync_copy(acc.at[pl.ds(obase, N_PAD // NS)],
                        out_hbm.at[lax.axis_index("c"),
                                   pl.ds(obase, N_PAD // NS)])

    return k(table, ls, ld, zacc)


def _mm_relu(x, W, b):
    def body(x_ref, w_ref, b_ref, o_ref):
        acc = jnp.dot(x_ref[...], w_ref[...], preferred_element_type=jnp.float32)
        o_ref[...] = jnp.maximum(acc + b_ref[...], 0.0)

    n = x.shape[0]
    return pl.pallas_call(
        body,
        out_shape=jax.ShapeDtypeStruct((n, W.shape[1]), jnp.float32),
    )(x, W, b.reshape(1, -1))


def _mm_relu_p(p, W, b):
    def body(p_ref, w_ref, b_ref, o_ref):
        h = p_ref[0] + p_ref[1]
        acc = jnp.dot(h, w_ref[...], preferred_element_type=jnp.float32)
        o_ref[...] = jnp.maximum(acc + b_ref[...], 0.0)

    n = p.shape[1]
    return pl.pallas_call(
        body,
        out_shape=jax.ShapeDtypeStruct((n, W.shape[1]), jnp.float32),
    )(p, W, b.reshape(1, -1))


def _final_linear(p2, Wf, bf):
    def body(p_ref, wf_ref, bf_ref, o_ref):
        h = p_ref[0] + p_ref[1]
        o_ref[...] = (jnp.dot(h, wf_ref[...],
                              preferred_element_type=jnp.float32)
                      + bf_ref[...])

    return pl.pallas_call(
        body,
        out_shape=jax.ShapeDtypeStruct((N_NODES, 1), jnp.float32),
    )(p2, Wf, bf.reshape(1, 1))


def kernel(x, edge_index, W1, b1, W2, b2, Wf, bf):
    src = edge_index[0].astype(jnp.int32)
    dst = edge_index[1].astype(jnp.int32)

    ls, ld, _ = _route(src, dst)
    zacc = jnp.zeros(((N_PAD + 128) // NS, D_HID), jnp.float32)

    m1 = _mm_relu(x, W1, b1)
    p1 = _ordered_segsum(m1, ls, ld, zacc)
    m2 = _mm_relu_p(p1[:, :N_NODES], W2, b2)
    p2 = _ordered_segsum(m2, ls, ld, zacc)
    return _final_linear(p2[:, :N_NODES], Wf, bf)
